# Initial kernel scaffold; baseline (speedup 1.0000x reference)
import jax
import jax.numpy as jnp
from jax.experimental import pallas as pl


def kernel(h, edge_index, edge_weight, W):
    raise NotImplementedError("write your pallas kernel here")


# speedup vs baseline: 3.6526x; 3.6526x over previous
import dataclasses
import functools

import jax
import jax.numpy as jnp
from jax import lax
from jax.experimental import pallas as pl
from jax.experimental.pallas import tpu as pltpu
from jax.experimental.pallas import tpu_sc as plsc

NUM_CORES = 2
NUM_SUBCORES = 16
LANES = 16


def _matmul_body(h_ref, w_ref, o_ref):
    o_ref[...] = lax.dot_general(
        h_ref[...], w_ref[...],
        dimension_numbers=(((1,), (1,)), ((), ())),
        preferred_element_type=jnp.float32,
        precision=lax.Precision.HIGHEST,
    )


def _linear(h, W):
    n, d_in = h.shape
    d_out = W.shape[0]
    blk = 1000 if n % 1000 == 0 else n
    return pl.pallas_call(
        _matmul_body,
        grid=(n // blk,),
        in_specs=[
            pl.BlockSpec((blk, d_in), lambda i: (i, 0)),
            pl.BlockSpec((d_out, d_in), lambda i: (0, 0)),
        ],
        out_specs=pl.BlockSpec((blk, d_out), lambda i: (i, 0)),
        out_shape=jax.ShapeDtypeStruct((n, d_out), jnp.float32),
    )(h, W)


def _add_body(p_ref, o_ref):
    o_ref[...] = p_ref[0] + p_ref[1]


def _sum_partials(partial):
    nc, n, d = partial.shape
    blk = 1000 if n % 1000 == 0 else n
    return pl.pallas_call(
        _add_body,
        grid=(n // blk,),
        in_specs=[pl.BlockSpec((nc, blk, d), lambda i: (0, i, 0))],
        out_specs=pl.BlockSpec((blk, d), lambda i: (i, 0)),
        out_shape=jax.ShapeDtypeStruct((n, d), jnp.float32),
    )(partial)


def _pick_chunk(edges_per_tile):
    for ch in range(128, 0, -8):
        if edges_per_tile % ch == 0:
            return ch
    return None


def _sc_aggregate(hl, dst, src, w):
    n, d = hl.shape
    e = dst.shape[0]
    nw = NUM_CORES * NUM_SUBCORES
    assert e % nw == 0 and n % NUM_SUBCORES == 0 and d % LANES == 0
    epw = e // nw
    ch = _pick_chunk(epw)
    n_chunks = epw // ch
    rows_u = (n // NUM_SUBCORES) // 8 * 8
    tail = n - rows_u * NUM_SUBCORES
    assert tail % 8 == 0 and tail <= ch

    mesh = plsc.VectorSubcoreMesh(core_axis_name="c", subcore_axis_name="s")
    cp = pltpu.CompilerParams()
    if "needs_layout_passes" in pltpu.CompilerParams.__dataclass_fields__:
        cp = dataclasses.replace(cp, needs_layout_passes=False)

    @functools.partial(
        pl.kernel,
        out_type=jax.ShapeDtypeStruct((NUM_CORES, n, d), jnp.float32),
        mesh=mesh,
        compiler_params=cp,
        scratch_types=[
            pltpu.VMEM((ch,), jnp.int32),
            pltpu.VMEM((ch,), jnp.int32),
            pltpu.VMEM((ch,), jnp.float32),
            pltpu.VMEM((ch, d), jnp.float32),
            pltpu.VMEM_SHARED((n, d), jnp.float32),
            pltpu.SemaphoreType.DMA,
        ],
    )
    def sc_kernel(hl_hbm, dst_hbm, src_hbm, w_hbm, out_hbm,
                  src_v, dst_v, w_v, rows_v, acc, sem):
        cid = lax.axis_index("c")
        sid = lax.axis_index("s")
        base = (cid * NUM_SUBCORES + sid) * epw

        zero16 = jnp.zeros((LANES,), jnp.float32)

        @pl.loop(0, ch)
        def _(r):
            @pl.loop(0, d, step=LANES)
            def _(j):
                rows_v[r, pl.ds(j, LANES)] = zero16

        nz = rows_u // ch
        rem = rows_u - nz * ch
        row0 = sid * rows_u

        @pl.loop(0, nz)
        def _(k):
            pltpu.sync_copy(rows_v, acc.at[pl.ds(row0 + k * ch, ch)])

        if rem:
            pltpu.sync_copy(rows_v.at[pl.ds(0, rem)],
                            acc.at[pl.ds(row0 + nz * ch, rem)])
        if tail:
            @pl.when(sid == NUM_SUBCORES - 1)
            def _():
                pltpu.sync_copy(rows_v.at[pl.ds(0, tail)],
                                acc.at[pl.ds(rows_u * NUM_SUBCORES, tail)])
        plsc.subcore_barrier()

        @pl.loop(0, n_chunks)
        def _(k):
            off = base + k * ch
            pltpu.sync_copy(src_hbm.at[pl.ds(off, ch)], src_v)
            pltpu.sync_copy(dst_hbm.at[pl.ds(off, ch)], dst_v)
            pltpu.sync_copy(w_hbm.at[pl.ds(off, ch)], w_v)
            pltpu.async_copy(hl_hbm.at[src_v], rows_v, sem).wait()

            @pl.loop(0, ch)
            def _(i):
                wb = plsc.load_gather(w_v, [jnp.full((LANES,), i, jnp.int32)])
                for j in range(d // LANES):
                    sl = pl.ds(j * LANES, LANES)
                    rows_v[i, sl] = rows_v[i, sl] * wb

            pltpu.sync_copy(rows_v, acc.at[dst_v], add=True)

        plsc.subcore_barrier()

        pltpu.sync_copy(acc.at[pl.ds(row0, rows_u)],
                        out_hbm.at[cid, pl.ds(row0, rows_u)])
        if tail:
            @pl.when(sid == NUM_SUBCORES - 1)
            def _():
                t0 = rows_u * NUM_SUBCORES
                pltpu.sync_copy(acc.at[pl.ds(t0, tail)],
                                out_hbm.at[cid, pl.ds(t0, tail)])

    return sc_kernel(hl, dst, src, w)


def kernel(h, edge_index, edge_weight, W):
    hl = _linear(h, W)
    partial = _sc_aggregate(hl, edge_index[0], edge_index[1], edge_weight)
    return _sum_partials(partial)


# speedup vs baseline: 6.4921x; 1.7774x over previous
import dataclasses
import functools

import jax
import jax.numpy as jnp
from jax import lax
from jax.experimental import pallas as pl
from jax.experimental.pallas import tpu as pltpu
from jax.experimental.pallas import tpu_sc as plsc

NUM_CORES = 2
NUM_SUBCORES = 16
LANES = 16
CH = 80
NROW = 3
NIDX = 6


def _matmul_body(h_ref, w_ref, o_ref):
    o_ref[...] = lax.dot_general(
        h_ref[...], w_ref[...],
        dimension_numbers=(((1,), (1,)), ((), ())),
        preferred_element_type=jnp.float32,
        precision=lax.Precision.HIGHEST,
    )


def _linear(h, W):
    n, d_in = h.shape
    d_out = W.shape[0]
    blk = 1000 if n % 1000 == 0 else n
    return pl.pallas_call(
        _matmul_body,
        grid=(n // blk,),
        in_specs=[
            pl.BlockSpec((blk, d_in), lambda i: (i, 0)),
            pl.BlockSpec((d_out, d_in), lambda i: (0, 0)),
        ],
        out_specs=pl.BlockSpec((blk, d_out), lambda i: (i, 0)),
        out_shape=jax.ShapeDtypeStruct((n, d_out), jnp.float32),
    )(h, W)


def _add_body(p_ref, o_ref):
    o_ref[...] = p_ref[0] + p_ref[1]


def _sum_partials(partial):
    nc, n, d = partial.shape
    blk = 1000 if n % 1000 == 0 else n
    return pl.pallas_call(
        _add_body,
        grid=(n // blk,),
        in_specs=[pl.BlockSpec((nc, blk, d), lambda i: (0, i, 0))],
        out_specs=pl.BlockSpec((blk, d), lambda i: (i, 0)),
        out_shape=jax.ShapeDtypeStruct((n, d), jnp.float32),
    )(partial)


def _sc_aggregate(hl, dst, src, w):
    n, d = hl.shape
    e = dst.shape[0]
    nw = NUM_CORES * NUM_SUBCORES
    assert d % LANES == 0
    unroll = NIDX
    epw = -(-e // (nw * CH * unroll)) * CH * unroll
    pad = nw * epw - e
    if pad:
        dst = jnp.concatenate([dst, jnp.zeros((pad,), dst.dtype)])
        src = jnp.concatenate([src, jnp.zeros((pad,), src.dtype)])
        w = jnp.concatenate([w, jnp.zeros((pad,), w.dtype)])
    n_chunks = epw // CH
    rows_u = (n // NUM_SUBCORES) // 8 * 8
    tail = n - rows_u * NUM_SUBCORES
    assert tail % 8 == 0 and tail <= CH

    mesh = plsc.VectorSubcoreMesh(core_axis_name="c", subcore_axis_name="s")
    cp = pltpu.CompilerParams()
    if "needs_layout_passes" in pltpu.CompilerParams.__dataclass_fields__:
        cp = dataclasses.replace(cp, needs_layout_passes=False)

    bcast_dnums = lax.GatherDimensionNumbers(
        offset_dims=(), collapsed_slice_dims=(0,), start_index_map=(0,))

    def _bcast_lane(vec, i):
        idx = jnp.reshape(lax.iota(jnp.int32, LANES) * 0 + i, (LANES, 1))
        return lax.gather(vec, idx, bcast_dnums, (1,),
                          mode=lax.GatherScatterMode.PROMISE_IN_BOUNDS)

    @functools.partial(
        pl.kernel,
        out_type=jax.ShapeDtypeStruct((NUM_CORES, n, d), jnp.float32),
        mesh=mesh,
        compiler_params=cp,
        scratch_types=(
            [pltpu.VMEM((CH, d), jnp.float32) for _ in range(NROW)]
            + [pltpu.VMEM((CH,), jnp.int32) for _ in range(NROW)]
            + [pltpu.VMEM((CH,), jnp.int32) for _ in range(NIDX)]
            + [pltpu.VMEM((CH,), jnp.float32) for _ in range(NIDX)]
            + [pltpu.VMEM_SHARED((n, d), jnp.float32)]
            + [pltpu.SemaphoreType.DMA] * (3 * NROW + 2 * NIDX)
        ),
    )
    def sc_kernel(hl_hbm, dst_hbm, src_hbm, w_hbm, out_hbm, *rest):
        rows = rest[:NROW]
        dst_v = rest[NROW:2 * NROW]
        src_v = rest[2 * NROW:2 * NROW + NIDX]
        w_v = rest[2 * NROW + NIDX:2 * NROW + 2 * NIDX]
        o = 2 * NROW + 2 * NIDX
        acc = rest[o]
        gsem = rest[o + 1:o + 1 + NROW]
        dsem = rest[o + 1 + NROW:o + 1 + 2 * NROW]
        csem = rest[o + 1 + 2 * NROW:o + 1 + 3 * NROW]
        isem = rest[o + 1 + 3 * NROW:o + 1 + 3 * NROW + NIDX]
        wsem = rest[o + 1 + 3 * NROW + NIDX:]
        cid = lax.axis_index("c")
        sid = lax.axis_index("s")
        base = (cid * NUM_SUBCORES + sid) * epw

        def start_idx(k, b):
            pltpu.async_copy(src_hbm.at[pl.ds(base + k * CH, CH)],
                             src_v[b], isem[b])
            pltpu.async_copy(w_hbm.at[pl.ds(base + k * CH, CH)],
                             w_v[b], wsem[b])

        def wait_idx(b):
            pltpu.make_async_copy(src_hbm.at[pl.ds(base, CH)],
                                  src_v[b], isem[b]).wait()
            pltpu.make_async_copy(w_hbm.at[pl.ds(base, CH)],
                                  w_v[b], wsem[b]).wait()

        def start_gather(k, b, ib):
            pltpu.async_copy(hl_hbm.at[src_v[ib]], rows[b], gsem[b])
            pltpu.async_copy(dst_hbm.at[pl.ds(base + k * CH, CH)],
                             dst_v[b], dsem[b])

        def wait_gather(b):
            pltpu.make_async_copy(hl_hbm.at[src_v[0]], rows[b],
                                  gsem[b]).wait()

        def start_scatter(b):
            pltpu.make_async_copy(dst_hbm.at[pl.ds(base, CH)],
                                  dst_v[b], dsem[b]).wait()
            pltpu.async_copy(rows[b], acc.at[dst_v[b]], csem[b], add=True)

        def wait_scatter(b):
            pltpu.make_async_copy(rows[b], acc.at[dst_v[b]],
                                  csem[b]).wait()

        def multiply(b, ib):
            @pl.loop(0, CH, step=LANES)
            def _(c):
                w16 = w_v[ib][pl.ds(c, LANES)]
                for i in range(LANES):
                    wb = _bcast_lane(w16, i)
                    for j in range(d // LANES):
                        sl = pl.ds(j * LANES, LANES)
                        rows[b][c + i, sl] = rows[b][c + i, sl] * wb

        zero16 = jnp.zeros((LANES,), jnp.float32)

        @pl.loop(0, CH)
        def _(r):
            @pl.loop(0, d, step=LANES)
            def _(j):
                rows[0][r, pl.ds(j, LANES)] = zero16

        nz = rows_u // CH
        rem = rows_u - nz * CH
        row0 = sid * rows_u

        @pl.loop(0, nz)
        def _(k):
            pltpu.sync_copy(rows[0], acc.at[pl.ds(row0 + k * CH, CH)])

        if rem:
            pltpu.sync_copy(rows[0].at[pl.ds(0, rem)],
                            acc.at[pl.ds(row0 + nz * CH, rem)])
        if tail:
            @pl.when(sid == NUM_SUBCORES - 1)
            def _():
                pltpu.sync_copy(rows[0].at[pl.ds(0, tail)],
                                acc.at[pl.ds(rows_u * NUM_SUBCORES, tail)])
        plsc.subcore_barrier()

        for k in range(NIDX - 1):
            start_idx(k, k)
        wait_idx(0)
        start_gather(0, 0, 0)

        @pl.loop(0, n_chunks // unroll)
        def _(p):
            for u in range(unroll):
                k = p * unroll + u
                b = u % NROW
                g = (u + 1) % NROW
                iu = (u + 1) % NIDX
                @pl.when(k + 1 < n_chunks)
                def _():
                    @pl.when(k + 1 >= NROW)
                    def _():
                        wait_scatter(g)
                    wait_idx(iu)
                    start_gather(k + 1, g, iu)
                wait_gather(b)
                multiply(b, u % NIDX)
                start_scatter(b)
                @pl.when(k + NIDX - 1 < n_chunks)
                def _():
                    start_idx(k + NIDX - 1, (u + NIDX - 1) % NIDX)

        for k in range(n_chunks - NROW, n_chunks):
            wait_scatter(k % NROW)
        plsc.subcore_barrier()

        pltpu.sync_copy(acc.at[pl.ds(row0, rows_u)],
                        out_hbm.at[cid, pl.ds(row0, rows_u)])
        if tail:
            @pl.when(sid == NUM_SUBCORES - 1)
            def _():
                t0 = rows_u * NUM_SUBCORES
                pltpu.sync_copy(acc.at[pl.ds(t0, tail)],
                                out_hbm.at[cid, pl.ds(t0, tail)])

    return sc_kernel(hl, dst, src, w)


def kernel(h, edge_index, edge_weight, W):
    hl = _linear(h, W)
    partial = _sc_aggregate(hl, edge_index[0], edge_index[1], edge_weight)
    return _sum_partials(partial)


# speedup vs baseline: 6.5729x; 1.0124x over previous
import dataclasses
import functools

import jax
import jax.numpy as jnp
from jax import lax
from jax.experimental import pallas as pl
from jax.experimental.pallas import tpu as pltpu
from jax.experimental.pallas import tpu_sc as plsc

NUM_CORES = 2
NUM_SUBCORES = 16
LANES = 16
CH = 112
NROW = 3
NIDX = 6


def _matmul_body(h_ref, w_ref, o_ref):
    o_ref[...] = lax.dot_general(
        h_ref[...], w_ref[...],
        dimension_numbers=(((1,), (1,)), ((), ())),
        preferred_element_type=jnp.float32,
        precision=lax.Precision.HIGHEST,
    )


def _linear(h, W):
    n, d_in = h.shape
    d_out = W.shape[0]
    blk = 1000 if n % 1000 == 0 else n
    return pl.pallas_call(
        _matmul_body,
        grid=(n // blk,),
        in_specs=[
            pl.BlockSpec((blk, d_in), lambda i: (i, 0)),
            pl.BlockSpec((d_out, d_in), lambda i: (0, 0)),
        ],
        out_specs=pl.BlockSpec((blk, d_out), lambda i: (i, 0)),
        out_shape=jax.ShapeDtypeStruct((n, d_out), jnp.float32),
    )(h, W)


def _add_body(p_ref, o_ref):
    o_ref[...] = p_ref[0] + p_ref[1]


def _sum_partials(partial):
    nc, n, d = partial.shape
    blk = 1000 if n % 1000 == 0 else n
    return pl.pallas_call(
        _add_body,
        grid=(n // blk,),
        in_specs=[pl.BlockSpec((nc, blk, d), lambda i: (0, i, 0))],
        out_specs=pl.BlockSpec((blk, d), lambda i: (i, 0)),
        out_shape=jax.ShapeDtypeStruct((n, d), jnp.float32),
    )(partial)


def _sc_aggregate(hl, dst, src, w):
    n, d = hl.shape
    e = dst.shape[0]
    nw = NUM_CORES * NUM_SUBCORES
    assert d % LANES == 0
    unroll = NIDX
    epw = -(-e // (nw * CH * unroll)) * CH * unroll
    pad = nw * epw - e
    if pad:
        dst = jnp.concatenate([dst, jnp.zeros((pad,), dst.dtype)])
        src = jnp.concatenate([src, jnp.zeros((pad,), src.dtype)])
        w = jnp.concatenate([w, jnp.zeros((pad,), w.dtype)])
    n_chunks = epw // CH
    rows_u = (n // NUM_SUBCORES) // 8 * 8
    tail = n - rows_u * NUM_SUBCORES
    assert tail % 8 == 0 and tail <= CH

    mesh = plsc.VectorSubcoreMesh(core_axis_name="c", subcore_axis_name="s")
    cp = pltpu.CompilerParams()
    if "needs_layout_passes" in pltpu.CompilerParams.__dataclass_fields__:
        cp = dataclasses.replace(cp, needs_layout_passes=False)

    bcast_dnums = lax.GatherDimensionNumbers(
        offset_dims=(), collapsed_slice_dims=(0,), start_index_map=(0,))

    def _bcast_lane(vec, i):
        idx = jnp.reshape(lax.iota(jnp.int32, LANES) * 0 + i, (LANES, 1))
        return lax.gather(vec, idx, bcast_dnums, (1,),
                          mode=lax.GatherScatterMode.PROMISE_IN_BOUNDS)

    @functools.partial(
        pl.kernel,
        out_type=jax.ShapeDtypeStruct((NUM_CORES, n, d), jnp.float32),
        mesh=mesh,
        compiler_params=cp,
        scratch_types=(
            [pltpu.VMEM((CH, d), jnp.float32) for _ in range(NROW)]
            + [pltpu.VMEM((CH,), jnp.int32) for _ in range(NROW)]
            + [pltpu.VMEM((CH,), jnp.int32) for _ in range(NIDX)]
            + [pltpu.VMEM((CH,), jnp.float32) for _ in range(NIDX)]
            + [pltpu.VMEM_SHARED((n, d), jnp.float32)]
            + [pltpu.SemaphoreType.DMA] * (3 * NROW + 2 * NIDX)
        ),
    )
    def sc_kernel(hl_hbm, dst_hbm, src_hbm, w_hbm, out_hbm, *rest):
        rows = rest[:NROW]
        dst_v = rest[NROW:2 * NROW]
        src_v = rest[2 * NROW:2 * NROW + NIDX]
        w_v = rest[2 * NROW + NIDX:2 * NROW + 2 * NIDX]
        o = 2 * NROW + 2 * NIDX
        acc = rest[o]
        gsem = rest[o + 1:o + 1 + NROW]
        dsem = rest[o + 1 + NROW:o + 1 + 2 * NROW]
        csem = rest[o + 1 + 2 * NROW:o + 1 + 3 * NROW]
        isem = rest[o + 1 + 3 * NROW:o + 1 + 3 * NROW + NIDX]
        wsem = rest[o + 1 + 3 * NROW + NIDX:]
        cid = lax.axis_index("c")
        sid = lax.axis_index("s")
        base = (cid * NUM_SUBCORES + sid) * epw

        def start_idx(k, b):
            pltpu.async_copy(src_hbm.at[pl.ds(base + k * CH, CH)],
                             src_v[b], isem[b])
            pltpu.async_copy(w_hbm.at[pl.ds(base + k * CH, CH)],
                             w_v[b], wsem[b])

        def wait_idx(b):
            pltpu.make_async_copy(src_hbm.at[pl.ds(base, CH)],
                                  src_v[b], isem[b]).wait()
            pltpu.make_async_copy(w_hbm.at[pl.ds(base, CH)],
                                  w_v[b], wsem[b]).wait()

        def start_gather(k, b, ib):
            pltpu.async_copy(hl_hbm.at[src_v[ib]], rows[b], gsem[b])
            pltpu.async_copy(dst_hbm.at[pl.ds(base + k * CH, CH)],
                             dst_v[b], dsem[b])

        def wait_gather(b):
            pltpu.make_async_copy(hl_hbm.at[src_v[0]], rows[b],
                                  gsem[b]).wait()

        def start_scatter(b):
            pltpu.make_async_copy(dst_hbm.at[pl.ds(base, CH)],
                                  dst_v[b], dsem[b]).wait()
            pltpu.async_copy(rows[b], acc.at[dst_v[b]], csem[b], add=True)

        def wait_scatter(b):
            pltpu.make_async_copy(rows[b], acc.at[dst_v[b]],
                                  csem[b]).wait()

        def multiply(b, ib):
            @pl.loop(0, CH, step=LANES)
            def _(c):
                w16 = w_v[ib][pl.ds(c, LANES)]
                for i in range(LANES):
                    wb = _bcast_lane(w16, i)
                    for j in range(d // LANES):
                        sl = pl.ds(j * LANES, LANES)
                        rows[b][c + i, sl] = rows[b][c + i, sl] * wb

        zero16 = jnp.zeros((LANES,), jnp.float32)

        @pl.loop(0, CH)
        def _(r):
            @pl.loop(0, d, step=LANES)
            def _(j):
                rows[0][r, pl.ds(j, LANES)] = zero16

        nz = rows_u // CH
        rem = rows_u - nz * CH
        row0 = sid * rows_u

        @pl.loop(0, nz)
        def _(k):
            pltpu.sync_copy(rows[0], acc.at[pl.ds(row0 + k * CH, CH)])

        if rem:
            pltpu.sync_copy(rows[0].at[pl.ds(0, rem)],
                            acc.at[pl.ds(row0 + nz * CH, rem)])
        if tail:
            @pl.when(sid == NUM_SUBCORES - 1)
            def _():
                pltpu.sync_copy(rows[0].at[pl.ds(0, tail)],
                                acc.at[pl.ds(rows_u * NUM_SUBCORES, tail)])
        plsc.subcore_barrier()

        for k in range(NIDX - 1):
            start_idx(k, k)
        wait_idx(0)
        start_gather(0, 0, 0)

        @pl.loop(0, n_chunks // unroll)
        def _(p):
            for u in range(unroll):
                k = p * unroll + u
                b = u % NROW
                g = (u + 1) % NROW
                iu = (u + 1) % NIDX
                @pl.when(k + 1 < n_chunks)
                def _():
                    @pl.when(k + 1 >= NROW)
                    def _():
                        wait_scatter(g)
                    wait_idx(iu)
                    start_gather(k + 1, g, iu)
                wait_gather(b)
                multiply(b, u % NIDX)
                start_scatter(b)
                @pl.when(k + NIDX - 1 < n_chunks)
                def _():
                    start_idx(k + NIDX - 1, (u + NIDX - 1) % NIDX)

        for k in range(n_chunks - NROW, n_chunks):
            wait_scatter(k % NROW)
        plsc.subcore_barrier()

        pltpu.sync_copy(acc.at[pl.ds(row0, rows_u)],
                        out_hbm.at[cid, pl.ds(row0, rows_u)])
        if tail:
            @pl.when(sid == NUM_SUBCORES - 1)
            def _():
                t0 = rows_u * NUM_SUBCORES
                pltpu.sync_copy(acc.at[pl.ds(t0, tail)],
                                out_hbm.at[cid, pl.ds(t0, tail)])

    return sc_kernel(hl, dst, src, w)


def kernel(h, edge_index, edge_weight, W):
    hl = _linear(h, W)
    partial = _sc_aggregate(hl, edge_index[0], edge_index[1], edge_weight)
    return _sum_partials(partial)


# speedup vs baseline: 6.7956x; 1.0339x over previous
import dataclasses
import functools

import jax
import jax.numpy as jnp
from jax import lax
from jax.experimental import pallas as pl
from jax.experimental.pallas import tpu as pltpu
from jax.experimental.pallas import tpu_sc as plsc

NUM_CORES = 2
NUM_SUBCORES = 16
LANES = 16
CH = 48
NROW = 6
NIDX = 6


def _matmul_body(h_ref, w_ref, o_ref):
    o_ref[...] = lax.dot_general(
        h_ref[...], w_ref[...],
        dimension_numbers=(((1,), (1,)), ((), ())),
        preferred_element_type=jnp.float32,
        precision=lax.Precision.HIGHEST,
    )


def _linear(h, W):
    n, d_in = h.shape
    d_out = W.shape[0]
    blk = 1000 if n % 1000 == 0 else n
    return pl.pallas_call(
        _matmul_body,
        grid=(n // blk,),
        in_specs=[
            pl.BlockSpec((blk, d_in), lambda i: (i, 0)),
            pl.BlockSpec((d_out, d_in), lambda i: (0, 0)),
        ],
        out_specs=pl.BlockSpec((blk, d_out), lambda i: (i, 0)),
        out_shape=jax.ShapeDtypeStruct((n, d_out), jnp.float32),
    )(h, W)


def _add_body(p_ref, o_ref):
    o_ref[...] = p_ref[0] + p_ref[1]


def _sum_partials(partial):
    nc, n, d = partial.shape
    blk = 1000 if n % 1000 == 0 else n
    return pl.pallas_call(
        _add_body,
        grid=(n // blk,),
        in_specs=[pl.BlockSpec((nc, blk, d), lambda i: (0, i, 0))],
        out_specs=pl.BlockSpec((blk, d), lambda i: (i, 0)),
        out_shape=jax.ShapeDtypeStruct((n, d), jnp.float32),
    )(partial)


def _sc_aggregate(hl, dst, src, w):
    n, d = hl.shape
    e = dst.shape[0]
    nw = NUM_CORES * NUM_SUBCORES
    assert d % LANES == 0
    unroll = NIDX
    epw = -(-e // (nw * CH * unroll)) * CH * unroll
    pad = nw * epw - e
    if pad:
        dst = jnp.concatenate([dst, jnp.zeros((pad,), dst.dtype)])
        src = jnp.concatenate([src, jnp.zeros((pad,), src.dtype)])
        w = jnp.concatenate([w, jnp.zeros((pad,), w.dtype)])
    n_chunks = epw // CH
    rows_u = (n // NUM_SUBCORES) // 8 * 8
    tail = n - rows_u * NUM_SUBCORES
    assert tail % 8 == 0 and tail <= CH

    mesh = plsc.VectorSubcoreMesh(core_axis_name="c", subcore_axis_name="s")
    cp = pltpu.CompilerParams()
    if "needs_layout_passes" in pltpu.CompilerParams.__dataclass_fields__:
        cp = dataclasses.replace(cp, needs_layout_passes=False)

    bcast_dnums = lax.GatherDimensionNumbers(
        offset_dims=(), collapsed_slice_dims=(0,), start_index_map=(0,))

    def _bcast_lane(vec, i):
        idx = jnp.reshape(lax.iota(jnp.int32, LANES) * 0 + i, (LANES, 1))
        return lax.gather(vec, idx, bcast_dnums, (1,),
                          mode=lax.GatherScatterMode.PROMISE_IN_BOUNDS)

    @functools.partial(
        pl.kernel,
        out_type=jax.ShapeDtypeStruct((NUM_CORES, n, d), jnp.float32),
        mesh=mesh,
        compiler_params=cp,
        scratch_types=(
            [pltpu.VMEM((CH, d), jnp.float32) for _ in range(NROW)]
            + [pltpu.VMEM((CH,), jnp.int32) for _ in range(NROW)]
            + [pltpu.VMEM((CH,), jnp.int32) for _ in range(NIDX)]
            + [pltpu.VMEM((CH,), jnp.float32) for _ in range(NIDX)]
            + [pltpu.VMEM_SHARED((n, d), jnp.float32)]
            + [pltpu.SemaphoreType.DMA] * (3 * NROW + 2 * NIDX)
        ),
    )
    def sc_kernel(hl_hbm, dst_hbm, src_hbm, w_hbm, out_hbm, *rest):
        rows = rest[:NROW]
        dst_v = rest[NROW:2 * NROW]
        src_v = rest[2 * NROW:2 * NROW + NIDX]
        w_v = rest[2 * NROW + NIDX:2 * NROW + 2 * NIDX]
        o = 2 * NROW + 2 * NIDX
        acc = rest[o]
        gsem = rest[o + 1:o + 1 + NROW]
        dsem = rest[o + 1 + NROW:o + 1 + 2 * NROW]
        csem = rest[o + 1 + 2 * NROW:o + 1 + 3 * NROW]
        isem = rest[o + 1 + 3 * NROW:o + 1 + 3 * NROW + NIDX]
        wsem = rest[o + 1 + 3 * NROW + NIDX:]
        cid = lax.axis_index("c")
        sid = lax.axis_index("s")
        base = (cid * NUM_SUBCORES + sid) * epw

        def start_idx(k, b):
            pltpu.async_copy(src_hbm.at[pl.ds(base + k * CH, CH)],
                             src_v[b], isem[b])
            pltpu.async_copy(w_hbm.at[pl.ds(base + k * CH, CH)],
                             w_v[b], wsem[b])

        def wait_idx(b):
            pltpu.make_async_copy(src_hbm.at[pl.ds(base, CH)],
                                  src_v[b], isem[b]).wait()
            pltpu.make_async_copy(w_hbm.at[pl.ds(base, CH)],
                                  w_v[b], wsem[b]).wait()

        def start_gather(k, b, ib):
            pltpu.async_copy(hl_hbm.at[src_v[ib]], rows[b], gsem[b])
            pltpu.async_copy(dst_hbm.at[pl.ds(base + k * CH, CH)],
                             dst_v[b], dsem[b])

        def wait_gather(b):
            pltpu.make_async_copy(hl_hbm.at[src_v[0]], rows[b],
                                  gsem[b]).wait()

        def start_scatter(b):
            pltpu.make_async_copy(dst_hbm.at[pl.ds(base, CH)],
                                  dst_v[b], dsem[b]).wait()
            pltpu.async_copy(rows[b], acc.at[dst_v[b]], csem[b], add=True)

        def wait_scatter(b):
            pltpu.make_async_copy(rows[b], acc.at[dst_v[b]],
                                  csem[b]).wait()

        def multiply(b, ib):
            @pl.loop(0, CH, step=LANES)
            def _(c):
                w16 = w_v[ib][pl.ds(c, LANES)]
                for i in range(LANES):
                    wb = _bcast_lane(w16, i)
                    for j in range(d // LANES):
                        sl = pl.ds(j * LANES, LANES)
                        rows[b][c + i, sl] = rows[b][c + i, sl] * wb

        zero16 = jnp.zeros((LANES,), jnp.float32)

        @pl.loop(0, CH)
        def _(r):
            @pl.loop(0, d, step=LANES)
            def _(j):
                rows[0][r, pl.ds(j, LANES)] = zero16

        nz = rows_u // CH
        rem = rows_u - nz * CH
        row0 = sid * rows_u

        @pl.loop(0, nz)
        def _(k):
            pltpu.sync_copy(rows[0], acc.at[pl.ds(row0 + k * CH, CH)])

        if rem:
            pltpu.sync_copy(rows[0].at[pl.ds(0, rem)],
                            acc.at[pl.ds(row0 + nz * CH, rem)])
        if tail:
            @pl.when(sid == NUM_SUBCORES - 1)
            def _():
                pltpu.sync_copy(rows[0].at[pl.ds(0, tail)],
                                acc.at[pl.ds(rows_u * NUM_SUBCORES, tail)])
        plsc.subcore_barrier()

        for k in range(NIDX - 1):
            start_idx(k, k)
        wait_idx(0)
        start_gather(0, 0, 0)
        wait_idx(1)
        start_gather(1, 1, 1)

        @pl.loop(0, n_chunks // unroll)
        def _(p):
            for u in range(unroll):
                k = p * unroll + u
                b = u % NROW
                g = (u + 2) % NROW
                iu = (u + 2) % NIDX
                @pl.when(k + 2 < n_chunks)
                def _():
                    @pl.when(k + 2 >= NROW)
                    def _():
                        wait_scatter(g)
                    wait_idx(iu)
                    start_gather(k + 2, g, iu)
                wait_gather(b)
                multiply(b, u % NIDX)
                start_scatter(b)
                @pl.when(k + NIDX - 1 < n_chunks)
                def _():
                    start_idx(k + NIDX - 1, (u + NIDX - 1) % NIDX)

        for k in range(n_chunks - NROW, n_chunks):
            wait_scatter(k % NROW)
        plsc.subcore_barrier()

        pltpu.sync_copy(acc.at[pl.ds(row0, rows_u)],
                        out_hbm.at[cid, pl.ds(row0, rows_u)])
        if tail:
            @pl.when(sid == NUM_SUBCORES - 1)
            def _():
                t0 = rows_u * NUM_SUBCORES
                pltpu.sync_copy(acc.at[pl.ds(t0, tail)],
                                out_hbm.at[cid, pl.ds(t0, tail)])

    return sc_kernel(hl, dst, src, w)


def kernel(h, edge_index, edge_weight, W):
    hl = _linear(h, W)
    partial = _sc_aggregate(hl, edge_index[0], edge_index[1], edge_weight)
    return _sum_partials(partial)
